# baseline (speedup 1.0000x reference)
import jax
import jax.numpy as jnp
from jax import lax
from jax.experimental import pallas as pl
from jax.experimental.pallas import tpu as pltpu
from jax.experimental.pallas import tpu_sc as plsc

NS = 16
CH = 32
DC = 128
WIN = 16
NB = 8


def _deg_body(rows_e, n_nodes):
    def body(dst_hbm, ew_hbm, out_hbm, dst_v, ew_v, deg_v):
        s = lax.axis_index("s")
        rpt = rows_e // NS
        pltpu.sync_copy(dst_hbm.at[pl.ds(s * rpt, rpt)], dst_v)
        pltpu.sync_copy(ew_hbm.at[pl.ds(s * rpt, rpt)], ew_v)

        z = jnp.zeros((16,), jnp.float32)

        def zero(i, _):
            deg_v[pl.ds(i * 16, 16)] = z
            return 0

        lax.fori_loop(0, n_nodes // 16, zero, 0)

        def upd(i, _):
            r = i // 8
            k = i % 8
            idx = dst_v[r, pl.ds(k * 16, 16)]
            w = ew_v[r, pl.ds(k * 16, 16)]
            plsc.addupdate_scatter(deg_v, [idx], w)
            return 0

        lax.fori_loop(0, rpt * 8, upd, 0)
        pltpu.sync_copy(deg_v, out_hbm.at[s])

    return body


def _coef_body(rows_e):
    def body(src_hbm, ew_hbm, dis_hbm, out_hbm, src_v, ew_v, dis_v):
        s = lax.axis_index("s")
        rpt = rows_e // NS
        pltpu.sync_copy(src_hbm.at[pl.ds(s * rpt, rpt)], src_v)
        pltpu.sync_copy(ew_hbm.at[pl.ds(s * rpt, rpt)], ew_v)
        pltpu.sync_copy(dis_hbm, dis_v)

        def upd(i, _):
            r = i // 8
            k = i % 8
            s16 = src_v[r, pl.ds(k * 16, 16)]
            w16 = ew_v[r, pl.ds(k * 16, 16)]
            d16 = plsc.load_gather(dis_v, [s16])
            ew_v[r, pl.ds(k * 16, 16)] = w16 * d16
            return 0

        lax.fori_loop(0, rpt * 8, upd, 0)
        pltpu.sync_copy(ew_v, out_hbm.at[pl.ds(s * rpt, rpt)])

    return body


def _prop_body(nslots, rows_e, n_nodes):
    stripe = n_nodes // NS

    def body(src_hbm, dst_hbm, cf_hbm, tbl_hbm, out_hbm,
             srcw0, srcw1, dstw0, dstw1, cw0, cw1, coef1,
             gb0, gb1, gb2, gb3, gb4, gb5, gb6, gb7, acc,
             gs0, gs1, gs2, gs3, gs4, gs5, gs6, gs7,
             ss0, ss1, ss2, ss3, ss4, ss5, ss6, ss7, wsa, wsb):
        s = lax.axis_index("s")
        ngrp = rows_e // NS
        row0 = s * ngrp
        nwin = ngrp // WIN
        _z = jnp.zeros((16,), jnp.float32)
        wbufs = ((srcw0, dstw0, cw0, wsa), (srcw1, dstw1, cw1, wsb))
        gbufs = (gb0, gb1, gb2, gb3, gb4, gb5, gb6, gb7)
        gsems = (gs0, gs1, gs2, gs3, gs4, gs5, gs6, gs7)
        ssems = (ss0, ss1, ss2, ss3, ss4, ss5, ss6, ss7)

        def stage(w, bi):
            base = row0 + w * WIN
            pltpu.async_copy(src_hbm.at[pl.ds(base, WIN)], wbufs[bi][0],
                             wbufs[bi][3])
            pltpu.async_copy(dst_hbm.at[pl.ds(base, WIN)], wbufs[bi][1],
                             wbufs[bi][3])
            pltpu.async_copy(cf_hbm.at[pl.ds(base, WIN)], wbufs[bi][2],
                             wbufs[bi][3])

        def stage_wait(bi):
            pltpu.make_async_copy(
                src_hbm.at[pl.ds(row0, WIN)], wbufs[bi][0],
                wbufs[bi][3]).wait()
            pltpu.make_async_copy(
                dst_hbm.at[pl.ds(row0, WIN)], wbufs[bi][1],
                wbufs[bi][3]).wait()
            pltpu.make_async_copy(
                cf_hbm.at[pl.ds(row0, WIN)], wbufs[bi][2],
                wbufs[bi][3]).wait()

        def scale(cf, g, rows):
            def cp(k, _):
                coef1[pl.ds(k * 16, 16)] = cf[g, pl.ds(k * 16, 16)]
                return 0

            lax.fori_loop(0, CH // 16, cp, 0)

            def sc_(b4, _):
                for u in range(4):
                    b = b4 * 4 + u
                    cb = plsc.load_gather(
                        coef1, [jnp.zeros((16,), jnp.int32) + b])
                    for j in range(DC // 16):
                        rows[b, pl.ds(j * 16, 16)] = (
                            rows[b, pl.ds(j * 16, 16)] * cb)
                return 0

            lax.fori_loop(0, CH // 4, sc_, 0)

        look = NB - 3

        def slot_body(slot, _):
            tbl = tbl_hbm.at[slot]

            def zb(i, _):
                r = i // (DC // 16)
                k = i % (DC // 16)
                gb7[r, pl.ds(k * 16, 16)] = _z
                return 0

            lax.fori_loop(0, CH * (DC // 16), zb, 0)
            done = 0
            while done < stripe:
                step = min(CH, stripe - done)
                pltpu.sync_copy(gb7.at[pl.ds(0, step)],
                                acc.at[pl.ds(s * stripe + done, step)])
                done += step
            plsc.subcore_barrier()

            stage(0, 0)
            stage_wait(0)
            for u in range(look, NB):
                pltpu.async_copy(gb7, acc.at[dstw0.at[0]], ssems[u],
                                 add=True)
            for u in range(look):
                pltpu.async_copy(tbl.at[srcw0.at[u]], gbufs[u], gsems[u])

            def win_block(w, bi):
                srcc, dstc, cfc, _ = wbufs[bi]
                srcn = wbufs[1 - bi][0]

                for c in range(WIN):
                    gb_ = gbufs[c % NB]
                    v = (c + look) % NB
                    la = c + look
                    pltpu.make_async_copy(
                        tbl.at[srcc.at[c]], gb_, gsems[c % NB]).wait()
                    scale(cfc, c, gb_)
                    pltpu.make_async_copy(
                        gbufs[v], acc.at[dstc.at[c]], ssems[v]).wait()
                    if la < WIN:
                        pltpu.async_copy(
                            tbl.at[srcc.at[la]], gbufs[v], gsems[v])
                    else:
                        if la == WIN:
                            @pl.when(w + 1 < nwin)
                            def _():
                                stage_wait(1 - bi)

                        @pl.when(w + 1 < nwin)
                        def _():
                            pltpu.async_copy(
                                tbl.at[srcn.at[la - WIN]], gbufs[v],
                                gsems[v])

                    pltpu.async_copy(
                        gb_, acc.at[dstc.at[c]], ssems[c % NB], add=True)

                    if c == 2:
                        @pl.when(w + 1 < nwin)
                        def _():
                            stage(w + 1, 1 - bi)

            def pair(k, _):
                win_block(2 * k, 0)
                win_block(2 * k + 1, 1)
                return 0

            lax.fori_loop(0, nwin // 2, pair, 0)
            for u in range(look, NB):
                pltpu.make_async_copy(gb7, acc.at[dstw0.at[0]],
                                      ssems[u]).wait()
            plsc.subcore_barrier()
            pltpu.sync_copy(acc.at[pl.ds(s * stripe, stripe)],
                            out_hbm.at[slot, s])
            return 0

        lax.fori_loop(0, nslots, slot_body, 0)

    return body


def _dis_kernel(degp_ref, out_ref):
    deg = jnp.sum(degp_ref[...], axis=0) + 1.0
    out_ref[...] = lax.rsqrt(deg)


def _mm1_kernel(x_ref, w_ref, out_ref):
    res = jnp.dot(x_ref[...], w_ref[...], preferred_element_type=jnp.float32)
    for j in range(out_ref.shape[0]):
        out_ref[j] = res[:, j * DC:(j + 1) * DC]


def _mm2_kernel(p1_ref, h1_ref, dis_ref, b1_ref, w2_ref, out_ref):
    d = dis_ref[...]
    hid = jax.nn.relu(d * p1_ref[...] + h1_ref[...] * (d * d)
                      + b1_ref[...][None, :])
    out_ref[...] = jnp.dot(hid, w2_ref[...],
                           preferred_element_type=jnp.float32)


def _final_kernel(p2_ref, h2_ref, dis_ref, b2_ref, out_ref):
    d = dis_ref[...]
    out_ref[...] = (d * p2_ref[...] + h2_ref[...] * (d * d)
                    + b2_ref[...][None, :])


def kernel(x, edge_index, edge_attr, W1, b1, W2, b2):
    n, d_in = x.shape
    d_hid = W1.shape[1]
    d_lat = W2.shape[1]
    e = edge_attr.shape[0]

    grain = NS * 2 * WIN * CH
    epad = ((e + grain - 1) // grain) * grain
    pad = epad - e
    src = jnp.concatenate([edge_index[0], jnp.zeros((pad,), edge_index.dtype)])
    dst = jnp.concatenate([edge_index[1], jnp.zeros((pad,), edge_index.dtype)])
    ew = jnp.concatenate([edge_attr, jnp.zeros((pad,), edge_attr.dtype)])
    rows_e = epad // CH
    src2 = src.reshape(rows_e, CH).astype(jnp.int32)
    dst2 = dst.reshape(rows_e, CH).astype(jnp.int32)
    ew2 = ew.reshape(rows_e, CH)
    rows_w = epad // 128
    src2w = src.reshape(rows_w, 128).astype(jnp.int32)
    dst2w = dst.reshape(rows_w, 128).astype(jnp.int32)
    ew2w = ew.reshape(rows_w, 128)

    mesh = plsc.VectorSubcoreMesh(
        core_axis_name="c", subcore_axis_name="s", num_cores=1)
    sc_params = pltpu.CompilerParams(needs_layout_passes=False)
    rpt = rows_w // NS

    deg_call = pl.kernel(
        _deg_body(rows_w, n),
        out_type=jax.ShapeDtypeStruct((NS, n), jnp.float32),
        mesh=mesh,
        compiler_params=sc_params,
        scratch_types=[
            pltpu.VMEM((rpt, 128), jnp.int32),
            pltpu.VMEM((rpt, 128), jnp.float32),
            pltpu.VMEM((n,), jnp.float32),
        ],
    )
    degp = deg_call(dst2w, ew2w)

    blk = 2000
    dis = pl.pallas_call(
        _dis_kernel,
        out_shape=jax.ShapeDtypeStruct((n,), jnp.float32),
    )(degp)

    coef_call = pl.kernel(
        _coef_body(rows_w),
        out_type=jax.ShapeDtypeStruct((rows_w, 128), jnp.float32),
        mesh=mesh,
        compiler_params=sc_params,
        scratch_types=[
            pltpu.VMEM((rpt, 128), jnp.int32),
            pltpu.VMEM((rpt, 128), jnp.float32),
            pltpu.VMEM((n,), jnp.float32),
        ],
    )
    cf2 = coef_call(src2w, ew2w, dis).reshape(rows_e, CH)

    nch1 = d_hid // DC
    t1 = pl.pallas_call(
        _mm1_kernel,
        grid=(n // blk,),
        in_specs=[
            pl.BlockSpec((blk, d_in), lambda r: (r, 0)),
            pl.BlockSpec((d_in, d_hid), lambda r: (0, 0)),
        ],
        out_specs=pl.BlockSpec((nch1, blk, DC), lambda r: (0, r, 0)),
        out_shape=jax.ShapeDtypeStruct((nch1, n, DC), jnp.float32),
    )(x, W1)

    stripe = n // NS

    def make_prop(nslots):
        return pl.kernel(
            _prop_body(nslots, rows_e, n),
            out_type=jax.ShapeDtypeStruct((nslots, NS, stripe, DC),
                                          jnp.float32),
            mesh=mesh,
            compiler_params=sc_params,
            scratch_types=(
                [pltpu.VMEM((WIN, CH), jnp.int32),
                 pltpu.VMEM((WIN, CH), jnp.int32),
                 pltpu.VMEM((WIN, CH), jnp.int32),
                 pltpu.VMEM((WIN, CH), jnp.int32),
                 pltpu.VMEM((WIN, CH), jnp.float32),
                 pltpu.VMEM((WIN, CH), jnp.float32),
                 pltpu.VMEM((CH,), jnp.float32)]
                + [pltpu.VMEM((CH, DC), jnp.float32) for _ in range(NB)]
                + [pltpu.VMEM_SHARED((n, DC), jnp.float32)]
                + [pltpu.SemaphoreType.DMA for _ in range(2 * NB + 2)]
            ),
        )

    prop1 = make_prop(nch1)
    p1 = prop1(src2, dst2, cf2, t1)
    p1cat = jnp.moveaxis(p1.reshape(nch1, n, DC), 0, 1).reshape(n, d_hid)
    h1cat = jnp.moveaxis(t1, 0, 1).reshape(n, d_hid)
    dis2d = dis.reshape(n, 1)

    h2 = pl.pallas_call(
        _mm2_kernel,
        grid=(n // blk,),
        in_specs=[
            pl.BlockSpec((blk, d_hid), lambda r: (r, 0)),
            pl.BlockSpec((blk, d_hid), lambda r: (r, 0)),
            pl.BlockSpec((blk, 1), lambda r: (r, 0)),
            pl.BlockSpec((d_hid,), lambda r: (0,)),
            pl.BlockSpec((d_hid, d_lat), lambda r: (0, 0)),
        ],
        out_specs=pl.BlockSpec((blk, d_lat), lambda r: (r, 0)),
        out_shape=jax.ShapeDtypeStruct((n, d_lat), jnp.float32),
    )(p1cat, h1cat, dis2d, b1, W2)

    prop2 = make_prop(1)
    p2 = prop2(src2, dst2, cf2, h2.reshape(1, n, d_lat))
    p2r = p2.reshape(n, d_lat)

    mu = pl.pallas_call(
        _final_kernel,
        grid=(n // blk,),
        in_specs=[
            pl.BlockSpec((blk, d_lat), lambda r: (r, 0)),
            pl.BlockSpec((blk, d_lat), lambda r: (r, 0)),
            pl.BlockSpec((blk, 1), lambda r: (r, 0)),
            pl.BlockSpec((d_lat,), lambda r: (0,)),
        ],
        out_specs=pl.BlockSpec((blk, d_lat), lambda r: (r, 0)),
        out_shape=jax.ShapeDtypeStruct((n, d_lat), jnp.float32),
    )(p2r, h2, dis2d, b2)
    return mu


# speedup vs baseline: 1.2044x; 1.2044x over previous
import jax
import jax.numpy as jnp
from jax import lax
from jax.experimental import pallas as pl
from jax.experimental.pallas import tpu as pltpu
from jax.experimental.pallas import tpu_sc as plsc

NS = 16
CH = 64
DC = 128
WIN = 8


def _deg_body(rows_e, n_nodes):
    def body(dst_hbm, ew_hbm, out_hbm, dst_v, ew_v, deg_v):
        s = lax.axis_index("s")
        rpt = rows_e // NS
        pltpu.sync_copy(dst_hbm.at[pl.ds(s * rpt, rpt)], dst_v)
        pltpu.sync_copy(ew_hbm.at[pl.ds(s * rpt, rpt)], ew_v)

        z = jnp.zeros((16,), jnp.float32)

        def zero(i, _):
            deg_v[pl.ds(i * 16, 16)] = z
            return 0

        lax.fori_loop(0, n_nodes // 16, zero, 0)

        def upd(i, _):
            r = i // (CH // 16)
            k = i % (CH // 16)
            idx = dst_v[r, pl.ds(k * 16, 16)]
            w = ew_v[r, pl.ds(k * 16, 16)]
            plsc.addupdate_scatter(deg_v, [idx], w)
            return 0

        lax.fori_loop(0, rpt * (CH // 16), upd, 0)
        pltpu.sync_copy(deg_v, out_hbm.at[s])

    return body


def _coef_body(rows_e):
    def body(src_hbm, ew_hbm, dis_hbm, out_hbm, src_v, ew_v, dis_v):
        s = lax.axis_index("s")
        rpt = rows_e // NS
        pltpu.sync_copy(src_hbm.at[pl.ds(s * rpt, rpt)], src_v)
        pltpu.sync_copy(ew_hbm.at[pl.ds(s * rpt, rpt)], ew_v)
        pltpu.sync_copy(dis_hbm, dis_v)

        def upd(i, _):
            r = i // (CH // 16)
            k = i % (CH // 16)
            s16 = src_v[r, pl.ds(k * 16, 16)]
            w16 = ew_v[r, pl.ds(k * 16, 16)]
            d16 = plsc.load_gather(dis_v, [s16])
            ew_v[r, pl.ds(k * 16, 16)] = w16 * d16
            return 0

        lax.fori_loop(0, rpt * (CH // 16), upd, 0)
        pltpu.sync_copy(ew_v, out_hbm.at[pl.ds(s * rpt, rpt)])

    return body


def _prop_body(nslots, rows_e, n_nodes):
    stripe = n_nodes // NS

    def body(src_hbm, dst_hbm, cf_hbm, tbl_hbm, out_hbm,
             srcw0, srcw1, dstw0, dstw1, cw0, cw1, coef1,
             gb0, gb1, gb2, gb3, acc,
             gs0, gs1, gs2, gs3, ss0, ss1, ss2, ss3, wsa, wsb):
        s = lax.axis_index("s")
        ngrp = rows_e // NS
        row0 = s * ngrp
        nwin = ngrp // WIN
        _z = jnp.zeros((16,), jnp.float32)
        wbufs = ((srcw0, dstw0, cw0, wsa), (srcw1, dstw1, cw1, wsb))
        gbufs = (gb0, gb1, gb2, gb3)
        gsems = (gs0, gs1, gs2, gs3)
        ssems = (ss0, ss1, ss2, ss3)

        def stage(w, bi):
            base = row0 + w * WIN
            pltpu.async_copy(src_hbm.at[pl.ds(base, WIN)], wbufs[bi][0],
                             wbufs[bi][3])
            pltpu.async_copy(dst_hbm.at[pl.ds(base, WIN)], wbufs[bi][1],
                             wbufs[bi][3])
            pltpu.async_copy(cf_hbm.at[pl.ds(base, WIN)], wbufs[bi][2],
                             wbufs[bi][3])

        def stage_wait(bi):
            pltpu.make_async_copy(
                src_hbm.at[pl.ds(row0, WIN)], wbufs[bi][0],
                wbufs[bi][3]).wait()
            pltpu.make_async_copy(
                dst_hbm.at[pl.ds(row0, WIN)], wbufs[bi][1],
                wbufs[bi][3]).wait()
            pltpu.make_async_copy(
                cf_hbm.at[pl.ds(row0, WIN)], wbufs[bi][2],
                wbufs[bi][3]).wait()

        def scale(cf, g, rows):
            def cp(k, _):
                coef1[pl.ds(k * 16, 16)] = cf[g, pl.ds(k * 16, 16)]
                return 0

            lax.fori_loop(0, CH // 16, cp, 0)

            def sc_(b4, _):
                for u in range(4):
                    b = b4 * 4 + u
                    cb = plsc.load_gather(
                        coef1, [jnp.zeros((16,), jnp.int32) + b])
                    for j in range(DC // 16):
                        rows[b, pl.ds(j * 16, 16)] = (
                            rows[b, pl.ds(j * 16, 16)] * cb)
                return 0

            lax.fori_loop(0, CH // 4, sc_, 0)

        for slot in range(nslots):
            tbl = tbl_hbm.at[slot]

            def zb(i, _):
                r = i // (DC // 16)
                k = i % (DC // 16)
                gb3[r, pl.ds(k * 16, 16)] = _z
                return 0

            lax.fori_loop(0, CH * (DC // 16), zb, 0)
            done = 0
            while done < stripe:
                step = min(CH, stripe - done)
                pltpu.sync_copy(gb3.at[pl.ds(0, step)],
                                acc.at[pl.ds(s * stripe + done, step)])
                done += step
            plsc.subcore_barrier()

            stage(0, 0)
            stage_wait(0)
            pltpu.async_copy(gb3, acc.at[dstw0.at[0]], ss3, add=True)
            for u in range(3):
                pltpu.async_copy(tbl.at[srcw0.at[u]], gbufs[u], gsems[u])

            def win_block(w, bi):
                srcc, dstc, cfc, _ = wbufs[bi]
                srcn = wbufs[1 - bi][0]

                for c in range(WIN):
                    gb_ = gbufs[c % 4]
                    v = (c + 3) % 4
                    la = c + 3
                    pltpu.make_async_copy(
                        tbl.at[srcc.at[c]], gb_, gsems[c % 4]).wait()
                    scale(cfc, c, gb_)
                    pltpu.make_async_copy(
                        gbufs[v], acc.at[dstc.at[c]], ssems[v]).wait()
                    if la < WIN:
                        pltpu.async_copy(
                            tbl.at[srcc.at[la]], gbufs[v], gsems[v])
                    else:
                        if la == WIN:
                            @pl.when(w + 1 < nwin)
                            def _():
                                stage_wait(1 - bi)

                        @pl.when(w + 1 < nwin)
                        def _():
                            pltpu.async_copy(
                                tbl.at[srcn.at[la - WIN]], gbufs[v],
                                gsems[v])

                    pltpu.async_copy(
                        gb_, acc.at[dstc.at[c]], ssems[c % 4], add=True)

                    if c == 0:
                        @pl.when(w + 1 < nwin)
                        def _():
                            stage(w + 1, 1 - bi)

            def pair(k, _):
                win_block(2 * k, 0)
                win_block(2 * k + 1, 1)
                return 0

            lax.fori_loop(0, nwin // 2, pair, 0)
            pltpu.make_async_copy(gb3, acc.at[dstw0.at[0]], ss3).wait()
            plsc.subcore_barrier()
            pltpu.sync_copy(acc.at[pl.ds(s * stripe, stripe)],
                            out_hbm.at[slot, s])

    return body


def _dis_kernel(degp_ref, out_ref):
    deg = jnp.sum(degp_ref[...], axis=0) + 1.0
    out_ref[...] = lax.rsqrt(deg)


def _mm1_kernel(x_ref, w_ref, out_ref):
    res = jnp.dot(x_ref[...], w_ref[...], preferred_element_type=jnp.float32)
    for j in range(out_ref.shape[0]):
        out_ref[j] = res[:, j * DC:(j + 1) * DC]


def _mm2_kernel(p1_ref, h1_ref, dis_ref, b1_ref, w2_ref, out_ref):
    d = dis_ref[...]
    hid = jax.nn.relu(d * p1_ref[...] + h1_ref[...] * (d * d)
                      + b1_ref[...][None, :])
    out_ref[...] = jnp.dot(hid, w2_ref[...],
                           preferred_element_type=jnp.float32)


def _final_kernel(p2_ref, h2_ref, dis_ref, b2_ref, out_ref):
    d = dis_ref[...]
    out_ref[...] = (d * p2_ref[...] + h2_ref[...] * (d * d)
                    + b2_ref[...][None, :])


def kernel(x, edge_index, edge_attr, W1, b1, W2, b2):
    n, d_in = x.shape
    d_hid = W1.shape[1]
    d_lat = W2.shape[1]
    e = edge_attr.shape[0]

    grain = NS * 2 * WIN * CH
    epad = ((e + grain - 1) // grain) * grain
    pad = epad - e
    src = jnp.concatenate([edge_index[0], jnp.zeros((pad,), edge_index.dtype)])
    dst = jnp.concatenate([edge_index[1], jnp.zeros((pad,), edge_index.dtype)])
    ew = jnp.concatenate([edge_attr, jnp.zeros((pad,), edge_attr.dtype)])
    rows_e = epad // CH
    src2 = src.reshape(rows_e, CH).astype(jnp.int32)
    dst2 = dst.reshape(rows_e, CH).astype(jnp.int32)
    ew2 = ew.reshape(rows_e, CH)

    mesh = plsc.VectorSubcoreMesh(
        core_axis_name="c", subcore_axis_name="s", num_cores=1)
    sc_params = pltpu.CompilerParams(needs_layout_passes=False)
    rpt = rows_e // NS

    deg_call = pl.kernel(
        _deg_body(rows_e, n),
        out_type=jax.ShapeDtypeStruct((NS, n), jnp.float32),
        mesh=mesh,
        compiler_params=sc_params,
        scratch_types=[
            pltpu.VMEM((rpt, CH), jnp.int32),
            pltpu.VMEM((rpt, CH), jnp.float32),
            pltpu.VMEM((n,), jnp.float32),
        ],
    )
    degp = deg_call(dst2, ew2)

    blk = 2000
    dis = pl.pallas_call(
        _dis_kernel,
        out_shape=jax.ShapeDtypeStruct((n,), jnp.float32),
    )(degp)

    coef_call = pl.kernel(
        _coef_body(rows_e),
        out_type=jax.ShapeDtypeStruct((rows_e, CH), jnp.float32),
        mesh=mesh,
        compiler_params=sc_params,
        scratch_types=[
            pltpu.VMEM((rpt, CH), jnp.int32),
            pltpu.VMEM((rpt, CH), jnp.float32),
            pltpu.VMEM((n,), jnp.float32),
        ],
    )
    cf2 = coef_call(src2, ew2, dis)

    nch1 = d_hid // DC
    t1 = pl.pallas_call(
        _mm1_kernel,
        grid=(n // blk,),
        in_specs=[
            pl.BlockSpec((blk, d_in), lambda r: (r, 0)),
            pl.BlockSpec((d_in, d_hid), lambda r: (0, 0)),
        ],
        out_specs=pl.BlockSpec((nch1, blk, DC), lambda r: (0, r, 0)),
        out_shape=jax.ShapeDtypeStruct((nch1, n, DC), jnp.float32),
    )(x, W1)

    stripe = n // NS

    def make_prop(nslots):
        return pl.kernel(
            _prop_body(nslots, rows_e, n),
            out_type=jax.ShapeDtypeStruct((nslots, NS, stripe, DC),
                                          jnp.float32),
            mesh=mesh,
            compiler_params=sc_params,
            scratch_types=(
                [pltpu.VMEM((WIN, CH), jnp.int32),
                 pltpu.VMEM((WIN, CH), jnp.int32),
                 pltpu.VMEM((WIN, CH), jnp.int32),
                 pltpu.VMEM((WIN, CH), jnp.int32),
                 pltpu.VMEM((WIN, CH), jnp.float32),
                 pltpu.VMEM((WIN, CH), jnp.float32),
                 pltpu.VMEM((CH,), jnp.float32)]
                + [pltpu.VMEM((CH, DC), jnp.float32) for _ in range(4)]
                + [pltpu.VMEM_SHARED((n, DC), jnp.float32)]
                + [pltpu.SemaphoreType.DMA for _ in range(10)]
            ),
        )

    prop1 = make_prop(nch1)
    p1 = prop1(src2, dst2, cf2, t1)
    p1cat = jnp.moveaxis(p1.reshape(nch1, n, DC), 0, 1).reshape(n, d_hid)
    h1cat = jnp.moveaxis(t1, 0, 1).reshape(n, d_hid)
    dis2d = dis.reshape(n, 1)

    h2 = pl.pallas_call(
        _mm2_kernel,
        grid=(n // blk,),
        in_specs=[
            pl.BlockSpec((blk, d_hid), lambda r: (r, 0)),
            pl.BlockSpec((blk, d_hid), lambda r: (r, 0)),
            pl.BlockSpec((blk, 1), lambda r: (r, 0)),
            pl.BlockSpec((d_hid,), lambda r: (0,)),
            pl.BlockSpec((d_hid, d_lat), lambda r: (0, 0)),
        ],
        out_specs=pl.BlockSpec((blk, d_lat), lambda r: (r, 0)),
        out_shape=jax.ShapeDtypeStruct((n, d_lat), jnp.float32),
    )(p1cat, h1cat, dis2d, b1, W2)

    prop2 = make_prop(1)
    p2 = prop2(src2, dst2, cf2, h2.reshape(1, n, d_lat))
    p2r = p2.reshape(n, d_lat)

    mu = pl.pallas_call(
        _final_kernel,
        grid=(n // blk,),
        in_specs=[
            pl.BlockSpec((blk, d_lat), lambda r: (r, 0)),
            pl.BlockSpec((blk, d_lat), lambda r: (r, 0)),
            pl.BlockSpec((blk, 1), lambda r: (r, 0)),
            pl.BlockSpec((d_lat,), lambda r: (0,)),
        ],
        out_specs=pl.BlockSpec((blk, d_lat), lambda r: (r, 0)),
        out_shape=jax.ShapeDtypeStruct((n, d_lat), jnp.float32),
    )(p2r, h2, dis2d, b2)
    return mu
